# initial kernel scaffold (unmeasured)
import jax
import jax.numpy as jnp
from jax import lax
from jax.experimental import pallas as pl
from jax.experimental.pallas import tpu as pltpu

N_DEV = 8


def kernel(x, w_mat, scale_x, scale_w):
    m_per, k = x.shape
    _, n_total = w_mat.shape
    n_per = n_total // N_DEV

    def body(x_ref, w_ref, sx_ref, sw_ref, out_ref,
             wbuf, send_buf, wsems, send_sems, recv_sems):
        my = lax.axis_index("i")

        barrier = pltpu.get_barrier_semaphore()
        for d in range(1, N_DEV):
            peer = lax.rem(my + d, N_DEV)
            pl.semaphore_signal(barrier, inc=1, device_id=(peer,),
                                device_id_type=pl.DeviceIdType.MESH)
        pl.semaphore_wait(barrier, N_DEV - 1)

        scale = sx_ref[0] * sw_ref[0]
        x_bf = x_ref[...].astype(jnp.bfloat16)

        order = list(range(1, N_DEV)) + [0]

        def w_copy(d, slot):
            tgt = lax.rem(my + d, N_DEV)
            return pltpu.make_async_copy(
                w_ref.at[:, pl.ds(tgt * n_per, n_per)],
                wbuf.at[slot],
                wsems.at[slot],
            )

        w_copy(order[0], 0).start()
        rdmas = {}
        for idx, d in enumerate(order):
            slot = idx % 2
            if idx + 1 < len(order):
                w_copy(order[idx + 1], (idx + 1) % 2).start()
            w_copy(d, slot).wait()
            wj = wbuf[slot].astype(jnp.bfloat16)
            acc = lax.dot_general(
                x_bf, wj, (((1,), (0,)), ((), ())),
                preferred_element_type=jnp.float32)
            y = acc * scale
            z = y * (1.0 / (1.0 + jnp.exp(-jnp.clip(y, -60.0, 60.0))))
            if d == 0:
                out_ref[pl.ds(my * m_per, m_per), :] = z
            else:
                send_buf[d - 1] = z
                tgt = lax.rem(my + d, N_DEV)
                rdma = pltpu.make_async_remote_copy(
                    src_ref=send_buf.at[d - 1],
                    dst_ref=out_ref.at[pl.ds(my * m_per, m_per)],
                    send_sem=send_sems.at[d - 1],
                    recv_sem=recv_sems.at[d - 1],
                    device_id=(tgt,),
                    device_id_type=pl.DeviceIdType.MESH,
                )
                rdma.start()
                rdmas[d] = rdma

        for d in range(1, N_DEV):
            src_dev = lax.rem(my - d + N_DEV, N_DEV)
            recv = pltpu.make_async_remote_copy(
                src_ref=send_buf.at[d - 1],
                dst_ref=out_ref.at[pl.ds(src_dev * m_per, m_per)],
                send_sem=send_sems.at[d - 1],
                recv_sem=recv_sems.at[d - 1],
                device_id=(src_dev,),
                device_id_type=pl.DeviceIdType.MESH,
            )
            recv.wait_recv()
        for d in range(1, N_DEV):
            rdmas[d].wait_send()

    return pl.pallas_call(
        body,
        out_shape=jax.ShapeDtypeStruct((N_DEV * m_per, n_per), jnp.float32),
        in_specs=[
            pl.BlockSpec(memory_space=pltpu.VMEM),
            pl.BlockSpec(memory_space=pltpu.ANY),
            pl.BlockSpec(memory_space=pltpu.SMEM),
            pl.BlockSpec(memory_space=pltpu.SMEM),
        ],
        out_specs=pl.BlockSpec(memory_space=pltpu.VMEM),
        scratch_shapes=[
            pltpu.VMEM((2, k, n_per), jnp.float32),
            pltpu.VMEM((N_DEV - 1, m_per, n_per), jnp.float32),
            pltpu.SemaphoreType.DMA((2,)),
            pltpu.SemaphoreType.DMA((N_DEV - 1,)),
            pltpu.SemaphoreType.DMA((N_DEV - 1,)),
        ],
        compiler_params=pltpu.CompilerParams(
            collective_id=0,
            vmem_limit_bytes=128 * 1024 * 1024,
        ),
    )(x, w_mat, scale_x, scale_w)


# baseline (device time: 112149 ns/iter reference)
import jax
import jax.numpy as jnp
from jax import lax
from jax.experimental import pallas as pl
from jax.experimental.pallas import tpu as pltpu

N_DEV = 8
H = 4


def kernel(x, w_mat, scale_x, scale_w):
    m_per, k = x.shape
    _, n_total = w_mat.shape
    n_per = n_total // N_DEV
    wc = n_per // H

    def body(x_ref, w_ref, sx_ref, sw_ref, out_ref,
             wbuf, send_buf, recv_buf, wsems, send_sems, recv_sems):
        my = lax.axis_index("i")

        barrier = pltpu.get_barrier_semaphore()
        for d in range(1, N_DEV):
            peer = lax.rem(my + d, N_DEV)
            pl.semaphore_signal(barrier, inc=1, device_id=(peer,),
                                device_id_type=pl.DeviceIdType.MESH)
        pl.semaphore_wait(barrier, N_DEV - 1)

        scale = sx_ref[0] * sw_ref[0]
        x_bf = x_ref[...].astype(jnp.bfloat16)

        order = list(range(1, N_DEV)) + [0]
        steps = [(d, h) for d in order for h in range(H)]

        def w_copy(d, h, slot):
            tgt = lax.rem(my + d, N_DEV)
            return pltpu.make_async_copy(
                w_ref.at[:, pl.ds(tgt * n_per + h * wc, wc)],
                wbuf.at[slot],
                wsems.at[slot],
            )

        w_copy(*steps[0], 0).start()
        rdmas = {}
        for si, (d, h) in enumerate(steps):
            slot = si % 2
            if si + 1 < len(steps):
                w_copy(*steps[si + 1], (si + 1) % 2).start()
            w_copy(d, h, slot).wait()
            wj = wbuf[slot].astype(jnp.bfloat16)
            acc = lax.dot_general(
                x_bf, wj, (((1,), (0,)), ((), ())),
                preferred_element_type=jnp.float32)
            y = acc * scale
            z = y * (1.0 / (1.0 + jnp.exp(-jnp.clip(y, -60.0, 60.0))))
            if d == 0:
                out_ref[pl.ds(my * m_per, m_per), pl.ds(h * wc, wc)] = z
            else:
                send_buf[d - 1, :, pl.ds(h * wc, wc)] = z.astype(jnp.bfloat16)
                if h == H - 1:
                    tgt = lax.rem(my + d, N_DEV)
                    rdma = pltpu.make_async_remote_copy(
                        src_ref=send_buf.at[d - 1],
                        dst_ref=recv_buf.at[d - 1],
                        send_sem=send_sems.at[d - 1],
                        recv_sem=recv_sems.at[d - 1],
                        device_id=(tgt,),
                        device_id_type=pl.DeviceIdType.MESH,
                    )
                    rdma.start()
                    rdmas[d] = rdma

        for d in range(1, N_DEV):
            src_dev = lax.rem(my - d + N_DEV, N_DEV)
            recv = pltpu.make_async_remote_copy(
                src_ref=send_buf.at[d - 1],
                dst_ref=recv_buf.at[d - 1],
                send_sem=send_sems.at[d - 1],
                recv_sem=recv_sems.at[d - 1],
                device_id=(src_dev,),
                device_id_type=pl.DeviceIdType.MESH,
            )
            recv.wait_recv()
            out_ref[pl.ds(src_dev * m_per, m_per), :] = (
                recv_buf[d - 1].astype(jnp.float32))
        for d in range(1, N_DEV):
            rdmas[d].wait_send()

    return pl.pallas_call(
        body,
        out_shape=jax.ShapeDtypeStruct((N_DEV * m_per, n_per), jnp.float32),
        in_specs=[
            pl.BlockSpec(memory_space=pltpu.VMEM),
            pl.BlockSpec(memory_space=pl.ANY),
            pl.BlockSpec(memory_space=pltpu.SMEM),
            pl.BlockSpec(memory_space=pltpu.SMEM),
        ],
        out_specs=pl.BlockSpec(memory_space=pltpu.VMEM),
        scratch_shapes=[
            pltpu.VMEM((2, k, wc), jnp.float32),
            pltpu.VMEM((N_DEV - 1, m_per, n_per), jnp.bfloat16),
            pltpu.VMEM((N_DEV - 1, m_per, n_per), jnp.bfloat16),
            pltpu.SemaphoreType.DMA((2,)),
            pltpu.SemaphoreType.DMA((N_DEV - 1,)),
            pltpu.SemaphoreType.DMA((N_DEV - 1,)),
        ],
        compiler_params=pltpu.CompilerParams(
            collective_id=0,
            vmem_limit_bytes=128 * 1024 * 1024,
        ),
    )(x, w_mat, scale_x, scale_w)


# device time: 101723 ns/iter; 1.1025x vs baseline; 1.1025x over previous
import jax
import jax.numpy as jnp
from jax import lax
from jax.experimental import pallas as pl
from jax.experimental.pallas import tpu as pltpu

N_DEV = 8
H = 4


def kernel(x, w_mat, scale_x, scale_w):
    m_per, k = x.shape
    _, n_total = w_mat.shape
    n_per = n_total // N_DEV
    wc = n_per // H

    def body(x_ref, w_ref, sx_ref, sw_ref, out_ref,
             wbuf, send_buf, recv_buf, wsems, send_sems, recv_sems):
        my = lax.axis_index("i")

        barrier = pltpu.get_barrier_semaphore()
        for d in range(1, N_DEV):
            peer = lax.rem(my + d, N_DEV)
            pl.semaphore_signal(barrier, inc=1, device_id=(peer,),
                                device_id_type=pl.DeviceIdType.MESH)
        pl.semaphore_wait(barrier, N_DEV - 1)

        scale = sx_ref[0] * sw_ref[0]
        x_bf = x_ref[...].astype(jnp.float8_e4m3fn)

        order = list(range(1, N_DEV)) + [0]
        steps = [(d, h) for d in order for h in range(H)]

        def w_copy(d, h, slot):
            tgt = lax.rem(my + d, N_DEV)
            return pltpu.make_async_copy(
                w_ref.at[:, pl.ds(tgt * n_per + h * wc, wc)],
                wbuf.at[slot],
                wsems.at[slot],
            )

        w_copy(*steps[0], 0).start()
        rdmas = {}
        for si, (d, h) in enumerate(steps):
            slot = si % 2
            if si + 1 < len(steps):
                w_copy(*steps[si + 1], (si + 1) % 2).start()
            w_copy(d, h, slot).wait()
            wj = wbuf[slot].astype(jnp.float8_e4m3fn)
            acc = lax.dot_general(
                x_bf, wj, (((1,), (0,)), ((), ())),
                preferred_element_type=jnp.float32)
            y = acc * scale
            z = y * (1.0 / (1.0 + jnp.exp(-jnp.clip(y, -60.0, 60.0))))
            if d == 0:
                out_ref[pl.ds(my * m_per, m_per), pl.ds(h * wc, wc)] = z
            else:
                send_buf[d - 1, :, pl.ds(h * wc, wc)] = z.astype(jnp.bfloat16)
                if h == H - 1:
                    tgt = lax.rem(my + d, N_DEV)
                    rdma = pltpu.make_async_remote_copy(
                        src_ref=send_buf.at[d - 1],
                        dst_ref=recv_buf.at[d - 1],
                        send_sem=send_sems.at[d - 1],
                        recv_sem=recv_sems.at[d - 1],
                        device_id=(tgt,),
                        device_id_type=pl.DeviceIdType.MESH,
                    )
                    rdma.start()
                    rdmas[d] = rdma

        for d in range(1, N_DEV):
            src_dev = lax.rem(my - d + N_DEV, N_DEV)
            recv = pltpu.make_async_remote_copy(
                src_ref=send_buf.at[d - 1],
                dst_ref=recv_buf.at[d - 1],
                send_sem=send_sems.at[d - 1],
                recv_sem=recv_sems.at[d - 1],
                device_id=(src_dev,),
                device_id_type=pl.DeviceIdType.MESH,
            )
            recv.wait_recv()
            out_ref[pl.ds(src_dev * m_per, m_per), :] = (
                recv_buf[d - 1].astype(jnp.float32))
        for d in range(1, N_DEV):
            rdmas[d].wait_send()

    return pl.pallas_call(
        body,
        out_shape=jax.ShapeDtypeStruct((N_DEV * m_per, n_per), jnp.float32),
        in_specs=[
            pl.BlockSpec(memory_space=pltpu.VMEM),
            pl.BlockSpec(memory_space=pl.ANY),
            pl.BlockSpec(memory_space=pltpu.SMEM),
            pl.BlockSpec(memory_space=pltpu.SMEM),
        ],
        out_specs=pl.BlockSpec(memory_space=pltpu.VMEM),
        scratch_shapes=[
            pltpu.VMEM((2, k, wc), jnp.float32),
            pltpu.VMEM((N_DEV - 1, m_per, n_per), jnp.bfloat16),
            pltpu.VMEM((N_DEV - 1, m_per, n_per), jnp.bfloat16),
            pltpu.SemaphoreType.DMA((2,)),
            pltpu.SemaphoreType.DMA((N_DEV - 1,)),
            pltpu.SemaphoreType.DMA((N_DEV - 1,)),
        ],
        compiler_params=pltpu.CompilerParams(
            collective_id=0,
            vmem_limit_bytes=128 * 1024 * 1024,
        ),
    )(x, w_mat, scale_x, scale_w)


# device time: 95948 ns/iter; 1.1689x vs baseline; 1.0602x over previous
import jax
import jax.numpy as jnp
from jax import lax
from jax.experimental import pallas as pl
from jax.experimental.pallas import tpu as pltpu

N_DEV = 8
H = 2


def kernel(x, w_mat, scale_x, scale_w):
    m_per, k = x.shape
    _, n_total = w_mat.shape
    n_per = n_total // N_DEV
    wc = n_per // H

    def body(x_ref, w_ref, sx_ref, sw_ref, out_ref,
             wbuf, send_buf, recv_buf, wsems, send_sems, recv_sems):
        my = lax.axis_index("i")

        barrier = pltpu.get_barrier_semaphore()
        for d in range(1, N_DEV):
            peer = lax.rem(my + d, N_DEV)
            pl.semaphore_signal(barrier, inc=1, device_id=(peer,),
                                device_id_type=pl.DeviceIdType.MESH)
        pl.semaphore_wait(barrier, N_DEV - 1)

        scale = sx_ref[0] * sw_ref[0]
        x_bf = x_ref[...].astype(jnp.float8_e4m3fn)

        order = list(range(1, N_DEV)) + [0]
        steps = [(d, h) for d in order for h in range(H)]

        def w_copy(d, h, slot):
            tgt = lax.rem(my + d, N_DEV)
            return pltpu.make_async_copy(
                w_ref.at[:, pl.ds(tgt * n_per + h * wc, wc)],
                wbuf.at[slot],
                wsems.at[slot],
            )

        w_copy(*steps[0], 0).start()
        rdmas = {}
        for si, (d, h) in enumerate(steps):
            slot = si % 2
            if si + 1 < len(steps):
                w_copy(*steps[si + 1], (si + 1) % 2).start()
            w_copy(d, h, slot).wait()
            wj = wbuf[slot].astype(jnp.float8_e4m3fn)
            acc = lax.dot_general(
                x_bf, wj, (((1,), (0,)), ((), ())),
                preferred_element_type=jnp.float32)
            y = acc * scale
            z = y * (1.0 / (1.0 + jnp.exp(-jnp.clip(y, -60.0, 60.0))))
            if d == 0:
                out_ref[pl.ds(my * m_per, m_per), pl.ds(h * wc, wc)] = z
            else:
                send_buf[d - 1, :, pl.ds(h * wc, wc)] = z.astype(jnp.bfloat16)
                if h == H - 1:
                    tgt = lax.rem(my + d, N_DEV)
                    rdma = pltpu.make_async_remote_copy(
                        src_ref=send_buf.at[d - 1],
                        dst_ref=recv_buf.at[d - 1],
                        send_sem=send_sems.at[d - 1],
                        recv_sem=recv_sems.at[d - 1],
                        device_id=(tgt,),
                        device_id_type=pl.DeviceIdType.MESH,
                    )
                    rdma.start()
                    rdmas[d] = rdma

        for d in range(1, N_DEV):
            src_dev = lax.rem(my - d + N_DEV, N_DEV)
            recv = pltpu.make_async_remote_copy(
                src_ref=send_buf.at[d - 1],
                dst_ref=recv_buf.at[d - 1],
                send_sem=send_sems.at[d - 1],
                recv_sem=recv_sems.at[d - 1],
                device_id=(src_dev,),
                device_id_type=pl.DeviceIdType.MESH,
            )
            recv.wait_recv()
            out_ref[pl.ds(src_dev * m_per, m_per), :] = (
                recv_buf[d - 1].astype(jnp.float32))
        for d in range(1, N_DEV):
            rdmas[d].wait_send()

    return pl.pallas_call(
        body,
        out_shape=jax.ShapeDtypeStruct((N_DEV * m_per, n_per), jnp.float32),
        in_specs=[
            pl.BlockSpec(memory_space=pltpu.VMEM),
            pl.BlockSpec(memory_space=pl.ANY),
            pl.BlockSpec(memory_space=pltpu.SMEM),
            pl.BlockSpec(memory_space=pltpu.SMEM),
        ],
        out_specs=pl.BlockSpec(memory_space=pltpu.VMEM),
        scratch_shapes=[
            pltpu.VMEM((2, k, wc), jnp.float32),
            pltpu.VMEM((N_DEV - 1, m_per, n_per), jnp.bfloat16),
            pltpu.VMEM((N_DEV - 1, m_per, n_per), jnp.bfloat16),
            pltpu.SemaphoreType.DMA((2,)),
            pltpu.SemaphoreType.DMA((N_DEV - 1,)),
            pltpu.SemaphoreType.DMA((N_DEV - 1,)),
        ],
        compiler_params=pltpu.CompilerParams(
            collective_id=0,
            vmem_limit_bytes=128 * 1024 * 1024,
        ),
    )(x, w_mat, scale_x, scale_w)


# device time: 85312 ns/iter; 1.3146x vs baseline; 1.1247x over previous
import os

import jax
import jax.numpy as jnp
from jax import lax
from jax.experimental import pallas as pl
from jax.experimental.pallas import tpu as pltpu

N_DEV = 8
H = 2
KVAR = os.environ.get("KVAR", "full")


def kernel(x, w_mat, scale_x, scale_w):
    m_per, k = x.shape
    _, n_total = w_mat.shape
    n_per = n_total // N_DEV
    wc = n_per // H

    def body(x_ref, w_ref, sx_ref, sw_ref, out_ref,
             wbuf, send_q, recv_q, send_s, recv_s,
             wsems, qsend_sems, qrecv_sems, ssend_sems, srecv_sems):
        my = lax.axis_index("i")

        if KVAR != "nocomm":
            barrier = pltpu.get_barrier_semaphore()
            for d in range(1, N_DEV):
                peer = lax.rem(my + d, N_DEV)
                pl.semaphore_signal(barrier, inc=1, device_id=(peer,),
                                    device_id_type=pl.DeviceIdType.MESH)
            pl.semaphore_wait(barrier, N_DEV - 1)

        scale = sx_ref[0] * sw_ref[0]
        x_q = x_ref[...].astype(jnp.float8_e4m3fn)

        order = list(range(1, N_DEV)) + [0]
        steps = [(d, h) for d in order for h in range(H)]

        def w_copy(d, h, slot):
            tgt = lax.rem(my + d, N_DEV)
            return pltpu.make_async_copy(
                w_ref.at[:, pl.ds(tgt * n_per + h * wc, wc)],
                wbuf.at[slot],
                wsems.at[slot],
            )

        def make_rdmas(d):
            tgt = lax.rem(my + d, N_DEV)
            data = pltpu.make_async_remote_copy(
                src_ref=send_q.at[d - 1],
                dst_ref=recv_q.at[d - 1],
                send_sem=qsend_sems.at[d - 1],
                recv_sem=qrecv_sems.at[d - 1],
                device_id=(tgt,),
                device_id_type=pl.DeviceIdType.MESH,
            )
            scl = pltpu.make_async_remote_copy(
                src_ref=send_s.at[d - 1],
                dst_ref=recv_s.at[d - 1],
                send_sem=ssend_sems.at[d - 1],
                recv_sem=srecv_sems.at[d - 1],
                device_id=(tgt,),
                device_id_type=pl.DeviceIdType.MESH,
            )
            return data, scl

        rdmas = {}
        if KVAR != "nocompute":
            w_copy(*steps[0], 0).start()
            for si, (d, h) in enumerate(steps):
                slot = si % 2
                if si + 1 < len(steps):
                    w_copy(*steps[si + 1], (si + 1) % 2).start()
                w_copy(d, h, slot).wait()
                wj = wbuf[slot].astype(jnp.float8_e4m3fn)
                acc = lax.dot_general(
                    x_q, wj, (((1,), (0,)), ((), ())),
                    preferred_element_type=jnp.float32)
                y = acc * scale
                z = y * (1.0 / (1.0 + jnp.exp(-jnp.clip(y, -60.0, 60.0))))
                if d == 0:
                    out_ref[pl.ds(my * m_per, m_per), pl.ds(h * wc, wc)] = z
                else:
                    s_h = jnp.maximum(
                        jnp.max(jnp.abs(z), axis=1, keepdims=True),
                        1e-30) * (1.0 / 127.0)
                    q = jnp.clip(jnp.round(z * (1.0 / s_h)), -127.0, 127.0)
                    send_q[d - 1, :, pl.ds(h * wc, wc)] = q.astype(jnp.int8)
                    send_s[d - 1, :, pl.ds(h, 1)] = s_h
                    if h == H - 1 and KVAR == "full":
                        data, scl = make_rdmas(d)
                        data.start()
                        scl.start()
                        rdmas[d] = (data, scl)
        else:
            for d in range(1, N_DEV):
                data, scl = make_rdmas(d)
                data.start()
                scl.start()
                rdmas[d] = (data, scl)

        if KVAR != "nocomm":
            for d in range(1, N_DEV):
                src_dev = lax.rem(my - d + N_DEV, N_DEV)
                data, scl = make_rdmas(d)
                data.wait_recv()
                scl.wait_recv()
                for h in range(H):
                    out_ref[pl.ds(src_dev * m_per, m_per),
                            pl.ds(h * wc, wc)] = (
                        recv_q[d - 1, :, pl.ds(h * wc, wc)].astype(
                            jnp.float32)
                        * recv_s[d - 1, :, pl.ds(h, 1)])
            for d in rdmas:
                rdmas[d][0].wait_send()
                rdmas[d][1].wait_send()
        else:
            for d in range(1, N_DEV):
                src_dev = lax.rem(my - d + N_DEV, N_DEV)
                out_ref[pl.ds(src_dev * m_per, m_per), :] = 0.0

    return pl.pallas_call(
        body,
        out_shape=jax.ShapeDtypeStruct((N_DEV * m_per, n_per), jnp.float32),
        in_specs=[
            pl.BlockSpec(memory_space=pltpu.VMEM),
            pl.BlockSpec(memory_space=pl.ANY),
            pl.BlockSpec(memory_space=pltpu.SMEM),
            pl.BlockSpec(memory_space=pltpu.SMEM),
        ],
        out_specs=pl.BlockSpec(memory_space=pltpu.VMEM),
        scratch_shapes=[
            pltpu.VMEM((2, k, wc), jnp.float32),
            pltpu.VMEM((N_DEV - 1, m_per, n_per), jnp.int8),
            pltpu.VMEM((N_DEV - 1, m_per, n_per), jnp.int8),
            pltpu.VMEM((N_DEV - 1, m_per, H), jnp.float32),
            pltpu.VMEM((N_DEV - 1, m_per, H), jnp.float32),
            pltpu.SemaphoreType.DMA((2,)),
            pltpu.SemaphoreType.DMA((N_DEV - 1,)),
            pltpu.SemaphoreType.DMA((N_DEV - 1,)),
            pltpu.SemaphoreType.DMA((N_DEV - 1,)),
            pltpu.SemaphoreType.DMA((N_DEV - 1,)),
        ],
        compiler_params=pltpu.CompilerParams(
            collective_id=None if KVAR == "nocomm" else 0,
            vmem_limit_bytes=128 * 1024 * 1024,
        ),
    )(x, w_mat, scale_x, scale_w)


# device time: 82558 ns/iter; 1.3584x vs baseline; 1.0334x over previous
import os

import jax
import jax.numpy as jnp
from jax import lax
from jax.experimental import pallas as pl
from jax.experimental.pallas import tpu as pltpu

N_DEV = 8
H = 2
KVAR = os.environ.get("KVAR", "full")


def kernel(x, w_mat, scale_x, scale_w):
    m_per, k = x.shape
    _, n_total = w_mat.shape
    n_per = n_total // N_DEV
    wc = n_per // H

    def body(x_ref, w_ref, sx_ref, sw_ref, out_ref,
             wbuf, send_q, recv_q, send_s, recv_s, stage,
             wsems, osems, qsend_sems, qrecv_sems, ssend_sems, srecv_sems):
        my = lax.axis_index("i")

        if KVAR != "nocomm":
            barrier = pltpu.get_barrier_semaphore()
            for d in range(1, N_DEV):
                peer = lax.rem(my + d, N_DEV)
                pl.semaphore_signal(barrier, inc=1, device_id=(peer,),
                                    device_id_type=pl.DeviceIdType.MESH)
            pl.semaphore_wait(barrier, N_DEV - 1)

        scale = sx_ref[0] * sw_ref[0]
        x_q = x_ref[...].astype(jnp.float8_e4m3fn)

        order = list(range(1, N_DEV)) + [0]
        steps = [(d, h) for d in order for h in range(H)]

        out_dmas = {}

        def flush_block(slot, row_dev):
            dma = pltpu.make_async_copy(
                stage.at[slot],
                out_ref.at[pl.ds(row_dev * m_per, m_per)],
                osems.at[slot],
            )
            dma.start()
            out_dmas[slot] = dma

        def reuse_slot(slot):
            if slot in out_dmas:
                out_dmas.pop(slot).wait()

        def w_copy(d, h, slot):
            tgt = lax.rem(my + d, N_DEV)
            return pltpu.make_async_copy(
                w_ref.at[:, pl.ds(tgt * n_per + h * wc, wc)],
                wbuf.at[slot],
                wsems.at[slot],
            )

        def make_rdmas(d):
            tgt = lax.rem(my + d, N_DEV)
            data = pltpu.make_async_remote_copy(
                src_ref=send_q.at[d - 1],
                dst_ref=recv_q.at[d - 1],
                send_sem=qsend_sems.at[d - 1],
                recv_sem=qrecv_sems.at[d - 1],
                device_id=(tgt,),
                device_id_type=pl.DeviceIdType.MESH,
            )
            scl = pltpu.make_async_remote_copy(
                src_ref=send_s.at[d - 1],
                dst_ref=recv_s.at[d - 1],
                send_sem=ssend_sems.at[d - 1],
                recv_sem=srecv_sems.at[d - 1],
                device_id=(tgt,),
                device_id_type=pl.DeviceIdType.MESH,
            )
            return data, scl

        rdmas = {}
        if KVAR != "nocompute":
            w_copy(*steps[0], 0).start()
            for si, (d, h) in enumerate(steps):
                slot = si % 2
                if si + 1 < len(steps):
                    w_copy(*steps[si + 1], (si + 1) % 2).start()
                w_copy(d, h, slot).wait()
                wj = wbuf[slot].astype(jnp.float8_e4m3fn)
                acc = lax.dot_general(
                    x_q, wj, (((1,), (0,)), ((), ())),
                    preferred_element_type=jnp.float32)
                y = acc * scale
                z = y * (1.0 / (1.0 + jnp.exp(-jnp.clip(y, -60.0, 60.0))))
                if d == 0:
                    if h == 0:
                        reuse_slot(0)
                    stage[0, :, pl.ds(h * wc, wc)] = z
                    if h == H - 1:
                        flush_block(0, my)
                else:
                    s_h = jnp.maximum(
                        jnp.max(jnp.abs(z), axis=1, keepdims=True),
                        1e-30) * (1.0 / 127.0)
                    q = jnp.clip(jnp.round(z * (1.0 / s_h)), -127.0, 127.0)
                    send_q[d - 1, :, pl.ds(h * wc, wc)] = q.astype(jnp.int8)
                    send_s[d - 1, :, pl.ds(h, 1)] = s_h
                    if h == H - 1 and KVAR == "full":
                        data, scl = make_rdmas(d)
                        data.start()
                        scl.start()
                        rdmas[d] = (data, scl)
        else:
            for d in range(1, N_DEV):
                data, scl = make_rdmas(d)
                data.start()
                scl.start()
                rdmas[d] = (data, scl)

        if KVAR != "nocomm":
            for d in range(1, N_DEV):
                src_dev = lax.rem(my - d + N_DEV, N_DEV)
                data, scl = make_rdmas(d)
                data.wait_recv()
                scl.wait_recv()
                slot = d % 2
                reuse_slot(slot)
                for h in range(H):
                    stage[slot, :, pl.ds(h * wc, wc)] = (
                        recv_q[d - 1, :, pl.ds(h * wc, wc)].astype(
                            jnp.float32)
                        * recv_s[d - 1, :, pl.ds(h, 1)])
                flush_block(slot, src_dev)
            for d in rdmas:
                rdmas[d][0].wait_send()
                rdmas[d][1].wait_send()
        for slot in list(out_dmas):
            out_dmas.pop(slot).wait()

    return pl.pallas_call(
        body,
        out_shape=jax.ShapeDtypeStruct((N_DEV * m_per, n_per), jnp.float32),
        in_specs=[
            pl.BlockSpec(memory_space=pltpu.VMEM),
            pl.BlockSpec(memory_space=pl.ANY),
            pl.BlockSpec(memory_space=pltpu.SMEM),
            pl.BlockSpec(memory_space=pltpu.SMEM),
        ],
        out_specs=pl.BlockSpec(memory_space=pl.ANY),
        scratch_shapes=[
            pltpu.VMEM((2, k, wc), jnp.float32),
            pltpu.VMEM((N_DEV - 1, m_per, n_per), jnp.int8),
            pltpu.VMEM((N_DEV - 1, m_per, n_per), jnp.int8),
            pltpu.VMEM((N_DEV - 1, m_per, H), jnp.float32),
            pltpu.VMEM((N_DEV - 1, m_per, H), jnp.float32),
            pltpu.VMEM((2, m_per, n_per), jnp.float32),
            pltpu.SemaphoreType.DMA((2,)),
            pltpu.SemaphoreType.DMA((2,)),
            pltpu.SemaphoreType.DMA((N_DEV - 1,)),
            pltpu.SemaphoreType.DMA((N_DEV - 1,)),
            pltpu.SemaphoreType.DMA((N_DEV - 1,)),
            pltpu.SemaphoreType.DMA((N_DEV - 1,)),
        ],
        compiler_params=pltpu.CompilerParams(
            collective_id=None if KVAR == "nocomm" else 0,
            vmem_limit_bytes=128 * 1024 * 1024,
        ),
    )(x, w_mat, scale_x, scale_w)


# device time: 75285 ns/iter; 1.4897x vs baseline; 1.0966x over previous
import os

import jax
import jax.numpy as jnp
from jax import lax
from jax.experimental import pallas as pl
from jax.experimental.pallas import tpu as pltpu

N_DEV = 8
H = 2
KVAR = os.environ.get("KVAR", "full")


def kernel(x, w_mat, scale_x, scale_w):
    m_per, k = x.shape
    _, n_total = w_mat.shape
    n_per = n_total // N_DEV
    wc = n_per // H

    def body(x_ref, w_ref, sx_ref, sw_ref, out_ref,
             wbuf, send_q, recv_q, send_s, recv_s, stage,
             wsems, osems, qsend_sems, qrecv_sems, ssend_sems, srecv_sems):
        my = lax.axis_index("i")

        if KVAR != "nocomm":
            barrier = pltpu.get_barrier_semaphore()
            for d in range(1, N_DEV):
                peer = lax.rem(my + d, N_DEV)
                pl.semaphore_signal(barrier, inc=1, device_id=(peer,),
                                    device_id_type=pl.DeviceIdType.MESH)
            pl.semaphore_wait(barrier, N_DEV - 1)

        scale = sx_ref[0] * sw_ref[0]
        x_q = x_ref[...].astype(jnp.float8_e4m3fn)

        order = list(range(1, N_DEV)) + [0]
        steps = [(d, h) for d in order for h in range(H)]

        out_dmas = {}

        def flush_block(slot, row_dev):
            dma = pltpu.make_async_copy(
                stage.at[slot],
                out_ref.at[pl.ds(row_dev * m_per, m_per)],
                osems.at[slot],
            )
            dma.start()
            out_dmas[slot] = dma

        def reuse_slot(slot):
            if slot in out_dmas:
                out_dmas.pop(slot).wait()

        def w_copy(d, h, slot):
            tgt = lax.rem(my + d, N_DEV)
            return pltpu.make_async_copy(
                w_ref.at[:, pl.ds(tgt * n_per + h * wc, wc)],
                wbuf.at[slot],
                wsems.at[slot],
            )

        def make_rdmas(d):
            tgt = lax.rem(my + d, N_DEV)
            data = pltpu.make_async_remote_copy(
                src_ref=send_q.at[d - 1],
                dst_ref=recv_q.at[d - 1],
                send_sem=qsend_sems.at[d - 1],
                recv_sem=qrecv_sems.at[d - 1],
                device_id=(tgt,),
                device_id_type=pl.DeviceIdType.MESH,
            )
            scl = pltpu.make_async_remote_copy(
                src_ref=send_s.at[d - 1],
                dst_ref=recv_s.at[d - 1],
                send_sem=ssend_sems.at[d - 1],
                recv_sem=srecv_sems.at[d - 1],
                device_id=(tgt,),
                device_id_type=pl.DeviceIdType.MESH,
            )
            return data, scl

        rdmas = {}
        if KVAR != "nocompute":
            w_copy(*steps[0], 0).start()
            for si, (d, h) in enumerate(steps):
                slot = si % 2
                if si + 1 < len(steps):
                    w_copy(*steps[si + 1], (si + 1) % 2).start()
                w_copy(d, h, slot).wait()
                wj = wbuf[slot].astype(jnp.float8_e4m3fn)
                acc = lax.dot_general(
                    x_q, wj, (((1,), (0,)), ((), ())),
                    preferred_element_type=jnp.float32)
                y = acc * scale
                z = y * (1.0 / (1.0 + jnp.exp(-jnp.clip(y, -60.0, 60.0))))
                if d == 0:
                    if h == 0:
                        reuse_slot(0)
                    stage[0, :, pl.ds(h * wc, wc)] = z
                    if h == H - 1:
                        flush_block(0, my)
                else:
                    s_h = jnp.maximum(
                        jnp.max(jnp.abs(z), axis=0, keepdims=True),
                        1e-30) * (1.0 / 127.0)
                    q = jnp.clip(jnp.round(z * (1.0 / s_h)), -127.0, 127.0)
                    send_q[d - 1, :, pl.ds(h * wc, wc)] = q.astype(jnp.int8)
                    send_s[d - 1, :, pl.ds(h * wc, wc)] = s_h
                    if h == H - 1 and KVAR == "full":
                        data, scl = make_rdmas(d)
                        data.start()
                        scl.start()
                        rdmas[d] = (data, scl)
        else:
            for d in range(1, N_DEV):
                data, scl = make_rdmas(d)
                data.start()
                scl.start()
                rdmas[d] = (data, scl)

        if KVAR != "nocomm":
            for d in range(1, N_DEV):
                src_dev = lax.rem(my - d + N_DEV, N_DEV)
                data, scl = make_rdmas(d)
                data.wait_recv()
                scl.wait_recv()
                slot = d % 2
                reuse_slot(slot)
                stage[slot] = (recv_q[d - 1].astype(jnp.float32)
                               * recv_s[d - 1])
                flush_block(slot, src_dev)
            for d in rdmas:
                rdmas[d][0].wait_send()
                rdmas[d][1].wait_send()
        for slot in list(out_dmas):
            out_dmas.pop(slot).wait()

    return pl.pallas_call(
        body,
        out_shape=jax.ShapeDtypeStruct((N_DEV * m_per, n_per), jnp.float32),
        in_specs=[
            pl.BlockSpec(memory_space=pltpu.VMEM),
            pl.BlockSpec(memory_space=pl.ANY),
            pl.BlockSpec(memory_space=pltpu.SMEM),
            pl.BlockSpec(memory_space=pltpu.SMEM),
        ],
        out_specs=pl.BlockSpec(memory_space=pl.ANY),
        scratch_shapes=[
            pltpu.VMEM((2, k, wc), jnp.float32),
            pltpu.VMEM((N_DEV - 1, m_per, n_per), jnp.int8),
            pltpu.VMEM((N_DEV - 1, m_per, n_per), jnp.int8),
            pltpu.VMEM((N_DEV - 1, 1, n_per), jnp.float32),
            pltpu.VMEM((N_DEV - 1, 1, n_per), jnp.float32),
            pltpu.VMEM((2, m_per, n_per), jnp.float32),
            pltpu.SemaphoreType.DMA((2,)),
            pltpu.SemaphoreType.DMA((2,)),
            pltpu.SemaphoreType.DMA((N_DEV - 1,)),
            pltpu.SemaphoreType.DMA((N_DEV - 1,)),
            pltpu.SemaphoreType.DMA((N_DEV - 1,)),
            pltpu.SemaphoreType.DMA((N_DEV - 1,)),
        ],
        compiler_params=pltpu.CompilerParams(
            collective_id=None if KVAR == "nocomm" else 0,
            vmem_limit_bytes=128 * 1024 * 1024,
        ),
    )(x, w_mat, scale_x, scale_w)


# device time: 74754 ns/iter; 1.5002x vs baseline; 1.0071x over previous
import os

import jax
import jax.numpy as jnp
from jax import lax
from jax.experimental import pallas as pl
from jax.experimental.pallas import tpu as pltpu

N_DEV = 8
H = 2
KVAR = os.environ.get("KVAR", "full")


def kernel(x, w_mat, scale_x, scale_w):
    m_per, k = x.shape
    _, n_total = w_mat.shape
    n_per = n_total // N_DEV
    wc = n_per // H

    def body(x_ref, w_ref, sx_ref, sw_ref, out_ref,
             wbuf, send_q, recv_q, send_s, recv_s, stage,
             wsems, osems, qsend_sems, qrecv_sems, ssend_sems, srecv_sems):
        my = lax.axis_index("i")

        scale = sx_ref[0] * sw_ref[0]
        x_q = x_ref[...].astype(jnp.float8_e4m3fn)

        order = list(range(1, N_DEV)) + [0]
        steps = [(d, h) for d in order for h in range(H)]

        out_dmas = {}

        def flush_block(slot, row_dev):
            dma = pltpu.make_async_copy(
                stage.at[slot],
                out_ref.at[pl.ds(row_dev * m_per, m_per)],
                osems.at[slot],
            )
            dma.start()
            out_dmas[slot] = dma

        def reuse_slot(slot):
            if slot in out_dmas:
                out_dmas.pop(slot).wait()

        def w_copy(d, h, slot):
            tgt = lax.rem(my + d, N_DEV)
            return pltpu.make_async_copy(
                w_ref.at[:, pl.ds(tgt * n_per + h * wc, wc)],
                wbuf.at[slot],
                wsems.at[slot],
            )

        def make_rdmas(d):
            tgt = lax.rem(my + d, N_DEV)
            data = pltpu.make_async_remote_copy(
                src_ref=send_q.at[d - 1],
                dst_ref=recv_q.at[d - 1],
                send_sem=qsend_sems.at[d - 1],
                recv_sem=qrecv_sems.at[d - 1],
                device_id=(tgt,),
                device_id_type=pl.DeviceIdType.MESH,
            )
            scl = pltpu.make_async_remote_copy(
                src_ref=send_s.at[d - 1],
                dst_ref=recv_s.at[d - 1],
                send_sem=ssend_sems.at[d - 1],
                recv_sem=srecv_sems.at[d - 1],
                device_id=(tgt,),
                device_id_type=pl.DeviceIdType.MESH,
            )
            return data, scl

        rdmas = {}
        if KVAR != "nocompute":
            w_copy(*steps[0], 0).start()
            w_copy(*steps[1], 1).start()
            for si, (d, h) in enumerate(steps):
                slot = si % 3
                if si + 2 < len(steps):
                    w_copy(*steps[si + 2], (si + 2) % 3).start()
                w_copy(d, h, slot).wait()
                wj = wbuf[slot].astype(jnp.float8_e4m3fn)
                acc = lax.dot_general(
                    x_q, wj, (((1,), (0,)), ((), ())),
                    preferred_element_type=jnp.float32)
                y = acc * scale
                z = y * (1.0 / (1.0 + jnp.exp(-jnp.clip(y, -60.0, 60.0))))
                if d == 0:
                    if h == 0:
                        reuse_slot(0)
                    stage[0, :, pl.ds(h * wc, wc)] = z
                    if h == H - 1:
                        flush_block(0, my)
                else:
                    s_h = jnp.maximum(
                        jnp.max(jnp.abs(z), axis=0, keepdims=True),
                        1e-30) * (1.0 / 127.0)
                    q = jnp.clip(jnp.round(z * (1.0 / s_h)), -127.0, 127.0)
                    send_q[d - 1, :, pl.ds(h * wc, wc)] = q.astype(jnp.int8)
                    send_s[d - 1, :, pl.ds(h * wc, wc)] = s_h
                    if h == H - 1 and KVAR == "full":
                        data, scl = make_rdmas(d)
                        data.start()
                        scl.start()
                        rdmas[d] = (data, scl)
        else:
            for d in range(1, N_DEV):
                data, scl = make_rdmas(d)
                data.start()
                scl.start()
                rdmas[d] = (data, scl)

        if KVAR != "nocomm":
            for d in range(1, N_DEV):
                src_dev = lax.rem(my - d + N_DEV, N_DEV)
                data, scl = make_rdmas(d)
                data.wait_recv()
                scl.wait_recv()
                slot = d % 2
                reuse_slot(slot)
                stage[slot] = (recv_q[d - 1].astype(jnp.float32)
                               * recv_s[d - 1])
                flush_block(slot, src_dev)
            for d in rdmas:
                rdmas[d][0].wait_send()
                rdmas[d][1].wait_send()
        for slot in list(out_dmas):
            out_dmas.pop(slot).wait()

    return pl.pallas_call(
        body,
        out_shape=jax.ShapeDtypeStruct((N_DEV * m_per, n_per), jnp.float32),
        in_specs=[
            pl.BlockSpec(memory_space=pltpu.VMEM),
            pl.BlockSpec(memory_space=pl.ANY),
            pl.BlockSpec(memory_space=pltpu.SMEM),
            pl.BlockSpec(memory_space=pltpu.SMEM),
        ],
        out_specs=pl.BlockSpec(memory_space=pl.ANY),
        scratch_shapes=[
            pltpu.VMEM((3, k, wc), jnp.float32),
            pltpu.VMEM((N_DEV - 1, m_per, n_per), jnp.int8),
            pltpu.VMEM((N_DEV - 1, m_per, n_per), jnp.int8),
            pltpu.VMEM((N_DEV - 1, 1, n_per), jnp.float32),
            pltpu.VMEM((N_DEV - 1, 1, n_per), jnp.float32),
            pltpu.VMEM((2, m_per, n_per), jnp.float32),
            pltpu.SemaphoreType.DMA((3,)),
            pltpu.SemaphoreType.DMA((2,)),
            pltpu.SemaphoreType.DMA((N_DEV - 1,)),
            pltpu.SemaphoreType.DMA((N_DEV - 1,)),
            pltpu.SemaphoreType.DMA((N_DEV - 1,)),
            pltpu.SemaphoreType.DMA((N_DEV - 1,)),
        ],
        compiler_params=pltpu.CompilerParams(
            vmem_limit_bytes=128 * 1024 * 1024,
        ),
    )(x, w_mat, scale_x, scale_w)
